# initial kernel scaffold (unmeasured)
import jax
import jax.numpy as jnp
from jax import lax
from jax.experimental import pallas as pl
from jax.experimental.pallas import tpu as pltpu


def kernel(x, dest):
    m, n = x.shape
    d_col = dest.reshape(m, 1)
    d_row = dest.reshape(1, m)

    def body(x_ref, dcol_ref, drow_ref, out_ref,
             send_buf, recv_buf, send_sem, recv_sem):
        my_x = lax.axis_index("x")
        my_y = lax.axis_index("y")
        my_z = lax.axis_index("z")
        partner = (1 - my_x, my_y, my_z)

        barrier_sem = pltpu.get_barrier_semaphore()
        pl.semaphore_signal(
            barrier_sem, inc=1,
            device_id=partner, device_id_type=pl.DeviceIdType.MESH,
        )
        pl.semaphore_wait(barrier_sem, 1)

        row = lax.broadcasted_iota(jnp.int32, (m, m), 0)
        col = lax.broadcasted_iota(jnp.int32, (m, m), 1)
        le = row <= col

        d_c = dcol_ref[:, :]
        d_r = drow_ref[:, :]
        keep_c = (d_c == my_x).astype(jnp.int32)
        send_c = 1 - keep_c
        keep_r = d_r == my_x
        send_r = jnp.logical_not(keep_r)

        p_send = jnp.sum(jnp.where(le, send_c, 0), axis=0, keepdims=True)
        p_keep = jnp.sum(jnp.where(le, keep_c, 0), axis=0, keepdims=True)

        c_keep = jnp.sum(keep_c)
        c_recv = m - c_keep
        keep_base = my_x * c_recv
        recv_base = (1 - my_x) * c_keep

        xb = x_ref[:, :].astype(jnp.bfloat16)

        s_mat = (send_r & ((p_send - 1) == row)).astype(jnp.bfloat16)
        send_buf[:, :] = jnp.dot(
            s_mat, xb, preferred_element_type=jnp.bfloat16
        )

        rdma = pltpu.make_async_remote_copy(
            src_ref=send_buf,
            dst_ref=recv_buf,
            send_sem=send_sem,
            recv_sem=recv_sem,
            device_id=partner,
            device_id_type=pl.DeviceIdType.MESH,
        )
        rdma.start()

        p_mat = (keep_r & ((p_keep - 1 + keep_base) == row)).astype(
            jnp.bfloat16
        )
        out_ref[:, :] = jnp.dot(p_mat, xb, preferred_element_type=jnp.float32)

        rdma.wait()

        q_mat = ((col < c_recv) & (row == recv_base + col)).astype(
            jnp.bfloat16
        )
        out_ref[:, :] += jnp.dot(
            q_mat, recv_buf[:, :], preferred_element_type=jnp.float32
        )

    return pl.pallas_call(
        body,
        out_shape=jax.ShapeDtypeStruct((m, n), jnp.float32),
        in_specs=[
            pl.BlockSpec(memory_space=pltpu.VMEM),
            pl.BlockSpec(memory_space=pltpu.VMEM),
            pl.BlockSpec(memory_space=pltpu.VMEM),
        ],
        out_specs=pl.BlockSpec(memory_space=pltpu.VMEM),
        scratch_shapes=[
            pltpu.VMEM((m, n), jnp.bfloat16),
            pltpu.VMEM((m, n), jnp.bfloat16),
            pltpu.SemaphoreType.DMA,
            pltpu.SemaphoreType.DMA,
        ],
        compiler_params=pltpu.CompilerParams(collective_id=0),
    )(x, d_col, d_row)


# baseline (device time: 10077 ns/iter reference)
import jax
import jax.numpy as jnp
from jax import lax
from jax.experimental import pallas as pl
from jax.experimental.pallas import tpu as pltpu


def kernel(x, dest):
    m, n = x.shape
    d_col = dest.reshape(m, 1)
    d_row = dest.reshape(1, m)

    def body(x_ref, dcol_ref, drow_ref, out_ref,
             send_buf, recv_buf, send_sem, recv_sem):
        my_x = lax.axis_index("x")
        my_y = lax.axis_index("y")
        my_z = lax.axis_index("z")
        partner = (1 - my_x, my_y, my_z)

        barrier_sem = pltpu.get_barrier_semaphore()
        pl.semaphore_signal(
            barrier_sem, inc=1,
            device_id=partner, device_id_type=pl.DeviceIdType.MESH,
        )
        pl.semaphore_wait(barrier_sem, 1)

        row = lax.broadcasted_iota(jnp.int32, (m, m), 0)
        col = lax.broadcasted_iota(jnp.int32, (m, m), 1)
        le = row <= col

        d_c = dcol_ref[:, :]
        d_r = drow_ref[:, :]
        keep_c = (d_c == my_x).astype(jnp.int32)
        send_c = 1 - keep_c
        keep_r = d_r == my_x
        send_r = jnp.logical_not(keep_r)

        p_send = jnp.sum(jnp.where(le, send_c, 0), axis=0, keepdims=True)
        p_keep = jnp.sum(jnp.where(le, keep_c, 0), axis=0, keepdims=True)

        c_keep = jnp.sum(keep_c)
        c_recv = m - c_keep
        keep_base = my_x * c_recv
        recv_base = (1 - my_x) * c_keep

        xb = x_ref[:, :].astype(jnp.bfloat16)

        s_mat = (send_r & ((p_send - 1) == row)).astype(jnp.bfloat16)
        send_buf[:, :] = jnp.dot(
            s_mat, xb, preferred_element_type=jnp.float32
        ).astype(jnp.bfloat16)

        rdma = pltpu.make_async_remote_copy(
            src_ref=send_buf,
            dst_ref=recv_buf,
            send_sem=send_sem,
            recv_sem=recv_sem,
            device_id=partner,
            device_id_type=pl.DeviceIdType.MESH,
        )
        rdma.start()

        p_mat = (keep_r & ((p_keep - 1 + keep_base) == row)).astype(
            jnp.bfloat16
        )
        out_ref[:, :] = jnp.dot(p_mat, xb, preferred_element_type=jnp.float32)

        rdma.wait()

        q_mat = ((col < c_recv) & (row == recv_base + col)).astype(
            jnp.bfloat16
        )
        out_ref[:, :] += jnp.dot(
            q_mat, recv_buf[:, :], preferred_element_type=jnp.float32
        )

    return pl.pallas_call(
        body,
        out_shape=jax.ShapeDtypeStruct((m, n), jnp.float32),
        in_specs=[
            pl.BlockSpec(memory_space=pltpu.VMEM),
            pl.BlockSpec(memory_space=pltpu.VMEM),
            pl.BlockSpec(memory_space=pltpu.VMEM),
        ],
        out_specs=pl.BlockSpec(memory_space=pltpu.VMEM),
        scratch_shapes=[
            pltpu.VMEM((m, n), jnp.bfloat16),
            pltpu.VMEM((m, n), jnp.bfloat16),
            pltpu.SemaphoreType.DMA,
            pltpu.SemaphoreType.DMA,
        ],
        compiler_params=pltpu.CompilerParams(collective_id=0),
    )(x, d_col, d_row)


# device time: 8654 ns/iter; 1.1644x vs baseline; 1.1644x over previous
import jax
import jax.numpy as jnp
from jax import lax
from jax.experimental import pallas as pl
from jax.experimental.pallas import tpu as pltpu

N_CHUNKS = 8


def kernel(x, dest):
    m, n = x.shape
    chunk = m // N_CHUNKS
    d_row = dest.reshape(1, m)

    def body(x_ref, drow_ref, out_ref, send_buf, recv_buf, send_sems, recv_sems):
        my_x = lax.axis_index("x")
        my_y = lax.axis_index("y")
        my_z = lax.axis_index("z")
        partner = (1 - my_x, my_y, my_z)

        def chunk_rdma(k):
            return pltpu.make_async_remote_copy(
                src_ref=send_buf.at[pl.ds(k * chunk, chunk)],
                dst_ref=recv_buf.at[pl.ds(k * chunk, chunk)],
                send_sem=send_sems.at[k],
                recv_sem=recv_sems.at[k],
                device_id=partner,
                device_id_type=pl.DeviceIdType.MESH,
            )

        recv_buf[:, :] = jnp.zeros((m, n), jnp.bfloat16)

        barrier_sem = pltpu.get_barrier_semaphore()
        pl.semaphore_signal(
            barrier_sem, inc=1,
            device_id=partner, device_id_type=pl.DeviceIdType.MESH,
        )

        row = lax.broadcasted_iota(jnp.int32, (m, m), 0)
        col = lax.broadcasted_iota(jnp.int32, (m, m), 1)
        col1 = lax.broadcasted_iota(jnp.int32, (1, m), 1)

        d_r = drow_ref[:, :]
        send_r = d_r != my_x

        le_bf = (row <= col).astype(jnp.bfloat16)
        p_send = jnp.dot(
            send_r.astype(jnp.bfloat16), le_bf,
            preferred_element_type=jnp.float32,
        ).astype(jnp.int32)

        c_send = p_send[0, m - 1]
        c_keep = m - c_send
        c_recv = c_send
        keep_base = my_x * c_recv
        recv_base = (1 - my_x) * c_keep

        s_tgt = jnp.where(send_r, p_send - 1, -1)
        k_tgt = jnp.where(send_r, -1, col1 - p_send + keep_base)

        xb = x_ref[:, :].astype(jnp.bfloat16)

        s_mat = (s_tgt == row).astype(jnp.bfloat16)
        send_buf[:, :] = jnp.dot(
            s_mat, xb, preferred_element_type=jnp.float32
        ).astype(jnp.bfloat16)

        pl.semaphore_wait(barrier_sem, 1)
        for k in range(N_CHUNKS):
            @pl.when(k * chunk < c_send)
            def _():
                chunk_rdma(k).start()

        p_mat = (k_tgt == row).astype(jnp.bfloat16)
        out_ref[:, :] = jnp.dot(p_mat, xb, preferred_element_type=jnp.float32)
        q_mat = (row == recv_base + col).astype(jnp.bfloat16)

        for k in range(N_CHUNKS):
            @pl.when(k * chunk < c_recv)
            def _():
                chunk_rdma(k).wait()

        out_ref[:, :] += jnp.dot(
            q_mat, recv_buf[:, :], preferred_element_type=jnp.float32
        )

    return pl.pallas_call(
        body,
        out_shape=jax.ShapeDtypeStruct((m, n), jnp.float32),
        in_specs=[
            pl.BlockSpec(memory_space=pltpu.VMEM),
            pl.BlockSpec(memory_space=pltpu.VMEM),
        ],
        out_specs=pl.BlockSpec(memory_space=pltpu.VMEM),
        scratch_shapes=[
            pltpu.VMEM((m, n), jnp.bfloat16),
            pltpu.VMEM((m, n), jnp.bfloat16),
            pltpu.SemaphoreType.DMA((N_CHUNKS,)),
            pltpu.SemaphoreType.DMA((N_CHUNKS,)),
        ],
        compiler_params=pltpu.CompilerParams(collective_id=0),
    )(x, d_row)
